# Initial kernel scaffold; baseline (speedup 1.0000x reference)
import jax
import jax.numpy as jnp
from jax.experimental import pallas as pl


def kernel(x, router_w, router_b, fc1_w, fc1_b, fc2_w, fc2_b):
    raise NotImplementedError("write your pallas kernel here")


# speedup vs baseline: 2.3841x; 2.3841x over previous
import functools

import jax
import jax.numpy as jnp
from jax.experimental import pallas as pl
from jax.experimental.pallas import tpu as pltpu

E = 16
TOP_K = 2
DIM = 1024
HID = 512
S = 2048
NEG_INF = -1e30


def _moe_dense_kernel(x_ref, rw_ref, rb_ref, fc1_ref, fc1b_ref, fc2_ref,
                      fc2b_ref, out_ref, aux_ref, route_ref):
    e = pl.program_id(0)

    @pl.when(e == 0)
    def _router():
        x = x_ref[...]
        logits = (jax.lax.dot_general(
            x, rw_ref[...], (((1,), (0,)), ((), ())),
            preferred_element_type=jnp.float32) + rb_ref[...]) * 10.0
        m = jnp.max(logits, axis=-1, keepdims=True)
        p = jnp.exp(logits - m)
        p = p / jnp.sum(p, axis=-1, keepdims=True)
        colsum = jnp.sum(p, axis=0)
        aux_ref[...] = (jnp.sum(colsum * colsum) / E * 1e-05).reshape(1, 1)
        lanes = jax.lax.broadcasted_iota(jnp.int32, logits.shape, 1)
        v1 = jnp.max(logits, axis=-1, keepdims=True)
        i1 = jnp.min(jnp.where(logits == v1, lanes, E), axis=-1, keepdims=True)
        masked = jnp.where(lanes == i1, NEG_INF, logits)
        v2 = jnp.max(masked, axis=-1, keepdims=True)
        i2 = jnp.min(jnp.where(masked == v2, lanes, E), axis=-1, keepdims=True)
        s = jnp.exp(v2 - v1)
        w1 = 1.0 / (1.0 + s)
        w2 = s / (1.0 + s)
        route_ref[:, 0:1] = w1
        route_ref[:, 1:2] = w2
        route_ref[:, 2:3] = i1.astype(jnp.float32)
        route_ref[:, 3:4] = i2.astype(jnp.float32)

    ef = e.astype(jnp.float32)
    gate = (route_ref[:, 0:1] * (route_ref[:, 2:3] == ef).astype(jnp.float32)
            + route_ref[:, 1:2] * (route_ref[:, 3:4] == ef).astype(jnp.float32))
    x = x_ref[...]
    h = jax.lax.dot_general(x, fc1_ref[0], (((1,), (0,)), ((), ())),
                            preferred_element_type=jnp.float32) + fc1b_ref[0]
    h = h * (1.0 / (1.0 + jnp.exp(-h)))
    y = jax.lax.dot_general(h, fc2_ref[0], (((1,), (0,)), ((), ())),
                            preferred_element_type=jnp.float32) + fc2b_ref[0]
    contrib = gate * y

    @pl.when(e == 0)
    def _init():
        out_ref[...] = contrib

    @pl.when(e != 0)
    def _acc():
        out_ref[...] = out_ref[...] + contrib


@jax.jit
def kernel(x, router_w, router_b, fc1_w, fc1_b, fc2_w, fc2_b):
    b, s, d = x.shape
    xf = x.reshape(-1, d)
    out, aux = pl.pallas_call(
        _moe_dense_kernel,
        grid=(E,),
        in_specs=[
            pl.BlockSpec((S, DIM), lambda e: (0, 0)),
            pl.BlockSpec((DIM, E), lambda e: (0, 0)),
            pl.BlockSpec((1, E), lambda e: (0, 0)),
            pl.BlockSpec((1, DIM, HID), lambda e: (e, 0, 0)),
            pl.BlockSpec((1, 1, HID), lambda e: (e, 0, 0)),
            pl.BlockSpec((1, HID, DIM), lambda e: (e, 0, 0)),
            pl.BlockSpec((1, 1, DIM), lambda e: (e, 0, 0)),
        ],
        out_specs=[
            pl.BlockSpec((S, DIM), lambda e: (0, 0)),
            pl.BlockSpec((1, 1), lambda e: (0, 0)),
        ],
        out_shape=[
            jax.ShapeDtypeStruct((S, DIM), jnp.float32),
            jax.ShapeDtypeStruct((1, 1), jnp.float32),
        ],
        scratch_shapes=[pltpu.VMEM((S, 4), jnp.float32)],
        compiler_params=pltpu.CompilerParams(
            dimension_semantics=("arbitrary",)),
    )(xf, router_w, router_b.reshape(1, E), fc1_w,
      fc1_b.reshape(E, 1, HID), fc2_w, fc2_b.reshape(E, 1, DIM))
    return out.reshape(b, s, d), aux.reshape(())
